# baseline (speedup 1.0000x reference)
import functools

import jax
import jax.numpy as jnp
from jax import lax
from jax.experimental import pallas as pl
from jax.experimental.pallas import tpu as pltpu
from jax.experimental.pallas import tpu_sc as plsc

B, S, D = 1, 2048, 768
H, KR = 12, 64
E, TOPK, DFF = 8, 2, 1536

SB = 256
TB = 512
NB = S // SB
NQ = S // TB

NPAIR = S * TOPK
BLKG = 128
MAXB = (NPAIR + E * (BLKG - 1)) // BLKG
P = MAXB * BLKG
CSB = 512


def _ln(h, g, b):
    mu = jnp.mean(h, axis=-1, keepdims=True)
    var = jnp.mean((h - mu) ** 2, axis=-1, keepdims=True)
    return (h - mu) * jax.lax.rsqrt(var + 1e-5) * g + b


def _elu1(x):
    return jnp.where(x > 0, x + 1.0, jnp.exp(x))


def _bdot(a, b):
    return jnp.dot(a.astype(jnp.bfloat16), b.astype(jnp.bfloat16),
                   preferred_element_type=jnp.float32)


def _bdot_t(a, b):
    return jax.lax.dot_general(
        a.astype(jnp.bfloat16), b.astype(jnp.bfloat16),
        dimension_numbers=(((0,), (0,)), ((), ())),
        preferred_element_type=jnp.float32)


def _attn_kv_kernel(x_ref, wk_ref, wv_ref, g_ref, b_ref,
                    kv_ref, ks_ref, kv_acc, ks_acc):
    bb = pl.program_id(0)

    @pl.when(bb == 0)
    def _init():
        kv_acc[...] = jnp.zeros_like(kv_acc)
        ks_acc[...] = jnp.zeros_like(ks_acc)

    h = _ln(x_ref[...], g_ref[...], b_ref[...])
    k = _bdot(h, wk_ref[...])
    v = _bdot(h, wv_ref[...])
    pk = _elu1(k)
    kv_acc[...] += _bdot_t(pk, v)
    ks_acc[0:1, :] += jnp.sum(pk, axis=0, keepdims=True)

    @pl.when(bb == NB - 1)
    def _fin():
        r = lax.broadcasted_iota(jnp.int32, (D, D), 0)
        c = lax.broadcasted_iota(jnp.int32, (D, D), 1)
        bd = (r // KR == c // KR).astype(jnp.float32)
        kv_ref[...] = kv_acc[...] * bd
        diag_ks = (r == c).astype(jnp.float32) * ks_acc[0:1, :]
        ks_ref[...] = jax.lax.dot_general(
            diag_ks, bd, (((1,), (0,)), ((), ())),
            preferred_element_type=jnp.float32)


def _attn_out_kernel(x_ref, wq_ref, wo_ref, g1_ref, b1_ref, g2_ref, b2_ref,
                     wr_ref, kv_ref, ks_ref, x1_ref, t_ref, logits_ref):
    x_blk = x_ref[...]
    h = _ln(x_blk, g1_ref[...], b1_ref[...])
    q = _bdot(h, wq_ref[...])
    pq = _elu1(q)
    num = _bdot(pq, kv_ref[...])
    den = _bdot(pq, ks_ref[...])
    attn_v = num / (den + 1e-6)
    x1 = x_blk + _bdot(attn_v, wo_ref[...])
    x1_ref[...] = x1
    t = _ln(x1, g2_ref[...], b2_ref[...])
    t_ref[...] = t
    logits_ref[...] = t @ wr_ref[...]


def _attention(x2d, Wq, Wk, Wv, Wo, g1, b1, g2, b2, Wr):
    full = lambda shape: pl.BlockSpec(shape, lambda bb: tuple(0 for _ in shape))
    kv, ks = pl.pallas_call(
        _attn_kv_kernel,
        grid=(NB,),
        in_specs=[
            pl.BlockSpec((SB, D), lambda bb: (bb, 0)),
            full((D, D)), full((D, D)), full((1, D)), full((1, D)),
        ],
        out_specs=[full((D, D)), full((D, D))],
        out_shape=[
            jax.ShapeDtypeStruct((D, D), jnp.float32),
            jax.ShapeDtypeStruct((D, D), jnp.float32),
        ],
        scratch_shapes=[
            pltpu.VMEM((D, D), jnp.float32),
            pltpu.VMEM((8, D), jnp.float32),
        ],
        compiler_params=pltpu.CompilerParams(
            dimension_semantics=("arbitrary",)),
    )(x2d, Wk, Wv, g1.reshape(1, D), b1.reshape(1, D))
    return pl.pallas_call(
        _attn_out_kernel,
        grid=(NB,),
        in_specs=[
            pl.BlockSpec((SB, D), lambda bb: (bb, 0)),
            full((D, D)), full((D, D)),
            full((1, D)), full((1, D)), full((1, D)), full((1, D)),
            full((D, E)),
            full((D, D)), full((D, D)),
        ],
        out_specs=[
            pl.BlockSpec((SB, D), lambda bb: (bb, 0)),
            pl.BlockSpec((SB, D), lambda bb: (bb, 0)),
            pl.BlockSpec((SB, E), lambda bb: (bb, 0)),
        ],
        out_shape=[
            jax.ShapeDtypeStruct((S, D), jnp.float32),
            jax.ShapeDtypeStruct((S, D), jnp.float32),
            jax.ShapeDtypeStruct((S, E), jnp.float32),
        ],
        compiler_params=pltpu.CompilerParams(
            dimension_semantics=("arbitrary",)),
    )(x2d, Wq, Wo, g1.reshape(1, D), b1.reshape(1, D),
      g2.reshape(1, D), b2.reshape(1, D), Wr, kv, ks)


def _router_kernel(logits_ref, tv0_ref, tv1_ref, pos0_ref, pos1_ref,
                   meta_ref):
    logits = logits_ref[...]
    m = jnp.max(logits, axis=1, keepdims=True)
    ex = jnp.exp(logits - m)
    p = ex / jnp.sum(ex, axis=1, keepdims=True)
    iota = lax.broadcasted_iota(jnp.int32, (S, E), 1)
    m0 = jnp.max(p, axis=1, keepdims=True)
    i0 = jnp.min(jnp.where(p == m0, iota, E), axis=1, keepdims=True)
    p1 = jnp.where(iota == i0, -1.0, p)
    m1 = jnp.max(p1, axis=1, keepdims=True)
    i1 = jnp.min(jnp.where(p1 == m1, iota, E), axis=1, keepdims=True)
    denom = m0 + m1
    tv0_ref[...] = m0 / denom
    tv1_ref[...] = m1 / denom

    oh0 = (iota == i0).astype(jnp.float32)
    oh1 = (iota == i1).astype(jnp.float32)
    r = lax.broadcasted_iota(jnp.int32, (CSB, CSB), 0)
    c = lax.broadcasted_iota(jnp.int32, (CSB, CSB), 1)
    ltri = (r > c).astype(jnp.float32)

    carry = jnp.zeros((1, E), jnp.float32)
    ranks = []
    for oh in (oh0, oh1):
        for bb in range(S // CSB):
            blk = oh[bb * CSB:(bb + 1) * CSB]
            ex_pre = jax.lax.dot_general(
                ltri, blk, (((1,), (0,)), ((), ()))) + carry
            ranks.append(jnp.sum(ex_pre * blk, axis=1, keepdims=True))
            carry = carry + jnp.sum(blk, axis=0, keepdims=True)
    cnt = carry
    cnt_i = cnt.astype(jnp.int32)
    pcnt_i = ((cnt_i + (BLKG - 1)) // BLKG) * BLKG
    pcnt = pcnt_i.astype(jnp.float32)
    re8 = lax.broadcasted_iota(jnp.int32, (E, E), 0)
    ce8 = lax.broadcasted_iota(jnp.int32, (E, E), 1)
    ltri8 = (re8 < ce8).astype(jnp.float32)
    off = jax.lax.dot_general(pcnt, ltri8, (((1,), (0,)), ((), ())))

    rank0 = jnp.concatenate(ranks[:S // CSB], axis=0)
    rank1 = jnp.concatenate(ranks[S // CSB:], axis=0)
    offg0 = jnp.sum(oh0 * off, axis=1, keepdims=True)
    offg1 = jnp.sum(oh1 * off, axis=1, keepdims=True)
    pos0_ref[...] = (offg0 + rank0).astype(jnp.int32)
    pos1_ref[...] = (offg1 + rank1).astype(jnp.int32)

    total_i = jnp.sum(pcnt).astype(jnp.int32)
    gstart = jnp.minimum(
        lax.broadcasted_iota(jnp.int32, (8, 128), 1) * BLKG,
        total_i - BLKG)
    acc = jnp.zeros((8, 128), jnp.int32)
    for e in range(E):
        sel = (lax.broadcasted_iota(jnp.int32, (1, E), 1) == e).astype(
            jnp.float32)
        off_e = jnp.sum(off * sel).astype(jnp.int32)
        pcnt_e = jnp.sum(pcnt * sel).astype(jnp.int32)
        in_e = (gstart >= off_e) & (gstart < off_e + pcnt_e)
        acc = acc + e * in_e.astype(jnp.int32)
    nact = (jnp.sum(pcnt) / BLKG).astype(jnp.int32)
    col = lax.broadcasted_iota(jnp.int32, (8, 128), 1)
    meta_ref[...] = jnp.where(col == MAXB, nact, acc)


def _router(logits):
    return pl.pallas_call(
        _router_kernel,
        grid=(1,),
        in_specs=[
            pl.BlockSpec((S, E), lambda i: (0, 0)),
        ],
        out_specs=[
            pl.BlockSpec((S, 1), lambda i: (0, 0)),
            pl.BlockSpec((S, 1), lambda i: (0, 0)),
            pl.BlockSpec((S, 1), lambda i: (0, 0)),
            pl.BlockSpec((S, 1), lambda i: (0, 0)),
            pl.BlockSpec((8, 128), lambda i: (0, 0)),
        ],
        out_shape=[
            jax.ShapeDtypeStruct((S, 1), jnp.float32),
            jax.ShapeDtypeStruct((S, 1), jnp.float32),
            jax.ShapeDtypeStruct((S, 1), jnp.int32),
            jax.ShapeDtypeStruct((S, 1), jnp.int32),
            jax.ShapeDtypeStruct((8, 128), jnp.int32),
        ],
    )(logits)


_TOK_PER_TILE = S // 32


@functools.lru_cache(maxsize=None)
def _sc_kernels():
    mesh = plsc.VectorSubcoreMesh(core_axis_name="c", subcore_axis_name="s")

    @functools.partial(
        pl.kernel, mesh=mesh,
        out_type=jax.ShapeDtypeStruct((P, D), jnp.float32),
        scratch_types=[
            pltpu.VMEM((_TOK_PER_TILE,), jnp.int32),
            pltpu.VMEM((_TOK_PER_TILE,), jnp.int32),
            pltpu.VMEM((_TOK_PER_TILE, D), jnp.float32),
        ],
    )
    def sc_dispatch(t_hbm, pos0_hbm, pos1_hbm, out_hbm,
                    idx0_v, idx1_v, rows_v):
        wid = lax.axis_index("s") * 2 + lax.axis_index("c")
        lo = wid * _TOK_PER_TILE
        sl = pl.ds(lo, _TOK_PER_TILE)
        pltpu.sync_copy(pos0_hbm.at[sl], idx0_v)
        pltpu.sync_copy(pos1_hbm.at[sl], idx1_v)
        pltpu.sync_copy(t_hbm.at[sl], rows_v)
        pltpu.sync_copy(rows_v, out_hbm.at[idx0_v])
        pltpu.sync_copy(rows_v, out_hbm.at[idx1_v])

    @functools.partial(
        pl.kernel, mesh=mesh,
        out_type=jax.ShapeDtypeStruct((2, S, D), jnp.float32),
        scratch_types=[
            pltpu.VMEM((_TOK_PER_TILE,), jnp.int32),
            pltpu.VMEM((_TOK_PER_TILE, D), jnp.float32),
            pltpu.SemaphoreType.DMA,
        ],
    )
    def sc_gather_pairs(po_hbm, pos0_hbm, pos1_hbm, out_hbm, idx_v, rows_v,
                        sem):
        wid = lax.axis_index("s") * 2 + lax.axis_index("c")
        lo = wid * _TOK_PER_TILE
        for k, pos_hbm in ((0, pos0_hbm), (1, pos1_hbm)):
            pltpu.sync_copy(pos_hbm.at[pl.ds(lo, _TOK_PER_TILE)], idx_v)
            pltpu.async_copy(po_hbm.at[idx_v], rows_v, sem).wait()
            pltpu.sync_copy(rows_v, out_hbm.at[k].at[pl.ds(lo, _TOK_PER_TILE)])

    return sc_dispatch, sc_gather_pairs


def _sc_dispatch(t, pos0, pos1):
    return _sc_kernels()[0](t, pos0, pos1)


def _sc_gather_pairs(packed_out, pos0, pos1):
    return _sc_kernels()[1](packed_out, pos0, pos1)


def _combine_kernel(x1_ref, g0_ref, g1_ref, tv0_ref, tv1_ref, out_ref):
    out_ref[...] = (x1_ref[...] + tv0_ref[...] * g0_ref[0]
                    + tv1_ref[...] * g1_ref[0])


def _combine(x1, gathered, tv0, tv1):
    return pl.pallas_call(
        _combine_kernel,
        grid=(NQ,),
        in_specs=[
            pl.BlockSpec((TB, D), lambda q: (q, 0)),
            pl.BlockSpec((1, TB, D), lambda q: (0, q, 0)),
            pl.BlockSpec((1, TB, D), lambda q: (1, q, 0)),
            pl.BlockSpec((TB, 1), lambda q: (q, 0)),
            pl.BlockSpec((TB, 1), lambda q: (q, 0)),
        ],
        out_specs=pl.BlockSpec((TB, D), lambda q: (q, 0)),
        out_shape=jax.ShapeDtypeStruct((S, D), jnp.float32),
        compiler_params=pltpu.CompilerParams(
            dimension_semantics=("arbitrary",)),
    )(x1, gathered, gathered, tv0, tv1)


def _gmm_kernel(meta_ref, pt_ref, w1_ref, b1_ref, w2_ref, b2_ref,
                out_ref):
    g = pl.program_id(0)

    @pl.when(g < meta_ref[MAXB])
    def _compute():
        h1 = jax.nn.gelu(_bdot(pt_ref[...], w1_ref[0]) + b1_ref[0])
        out_ref[...] = _bdot(h1, w2_ref[0]) + b2_ref[0]


def _grouped_mm(meta_row, packed_t, W1, b1, W2, b2):
    grid_spec = pltpu.PrefetchScalarGridSpec(
        num_scalar_prefetch=1,
        grid=(MAXB,),
        in_specs=[
            pl.BlockSpec((BLKG, D),
                         lambda g, mr: (jnp.minimum(g, mr[MAXB] - 1), 0)),
            pl.BlockSpec((1, D, DFF), lambda g, mr: (mr[g], 0, 0)),
            pl.BlockSpec((1, 1, DFF), lambda g, mr: (mr[g], 0, 0)),
            pl.BlockSpec((1, DFF, D), lambda g, mr: (mr[g], 0, 0)),
            pl.BlockSpec((1, 1, D), lambda g, mr: (mr[g], 0, 0)),
        ],
        out_specs=pl.BlockSpec((BLKG, D), lambda g, mr: (g, 0)),
    )
    return pl.pallas_call(
        _gmm_kernel,
        grid_spec=grid_spec,
        out_shape=jax.ShapeDtypeStruct((P, D), jnp.float32),
        compiler_params=pltpu.CompilerParams(
            dimension_semantics=("arbitrary",)),
    )(meta_row, packed_t, W1,
      b1.reshape(E, 1, DFF), W2, b2.reshape(E, 1, D))


@jax.jit
def kernel(x, Wq, Wk, Wv, Wo, ln1_g, ln1_b, ln2_g, ln2_b, Wr, W1, b1, W2, b2):
    x2d = x.reshape(S, D)
    x1, t, logits = _attention(x2d, Wq, Wk, Wv, Wo, ln1_g, ln1_b,
                               ln2_g, ln2_b, Wr)
    tv0, tv1, pos0, pos1, meta = _router(logits)
    pos0 = pos0.reshape(S)
    pos1 = pos1.reshape(S)
    packed_t = _sc_dispatch(t, pos0, pos1)
    packed_out = _grouped_mm(meta[0], packed_t, W1, b1, W2, b2)
    gathered = _sc_gather_pairs(packed_out, pos0, pos1)
    out = _combine(x1, gathered, tv0, tv1)
    return out.reshape(B, S, D)


# speedup vs baseline: 1.0869x; 1.0869x over previous
import functools

import jax
import jax.numpy as jnp
from jax import lax
from jax.experimental import pallas as pl
from jax.experimental.pallas import tpu as pltpu
from jax.experimental.pallas import tpu_sc as plsc

B, S, D = 1, 2048, 768
H, KR = 12, 64
E, TOPK, DFF = 8, 2, 1536

SB = 256
TB = 512
NB = S // SB
NQ = S // TB

NPAIR = S * TOPK
BLKG = 256
MAXB = (NPAIR + E * (BLKG - 1)) // BLKG
P = 6144
CSB = 512


def _ln(h, g, b):
    mu = jnp.mean(h, axis=-1, keepdims=True)
    var = jnp.mean((h - mu) ** 2, axis=-1, keepdims=True)
    return (h - mu) * jax.lax.rsqrt(var + 1e-5) * g + b


def _elu1(x):
    return jnp.where(x > 0, x + 1.0, jnp.exp(x))


def _bdot(a, b):
    return jnp.dot(a.astype(jnp.bfloat16), b.astype(jnp.bfloat16),
                   preferred_element_type=jnp.float32)


def _bdot_t(a, b):
    return jax.lax.dot_general(
        a.astype(jnp.bfloat16), b.astype(jnp.bfloat16),
        dimension_numbers=(((0,), (0,)), ((), ())),
        preferred_element_type=jnp.float32)


def _attn_kernel(x_ref, wq_ref, wk_ref, wv_ref, wo_ref,
                 g1_ref, b1_ref, g2_ref, b2_ref, wr_ref,
                 x1_ref, t_ref, logits_ref, kv_bd, ks_bd, ks_acc):
    p = pl.program_id(0)
    bb = pl.program_id(1)

    @pl.when((p == 0) & (bb == 0))
    def _init():
        kv_bd[...] = jnp.zeros_like(kv_bd)
        ks_acc[...] = jnp.zeros_like(ks_acc)

    x_blk = x_ref[...]
    h = _ln(x_blk, g1_ref[...], b1_ref[...])

    @pl.when(p == 0)
    def _acc():
        k = _bdot(h, wk_ref[...])
        v = _bdot(h, wv_ref[...])
        pk = _elu1(k)
        kv_bd[...] += _bdot_t(pk, v)
        ks_acc[0:1, :] += jnp.sum(pk, axis=0, keepdims=True)

    @pl.when((p == 0) & (bb == NB - 1))
    def _fin():
        r = lax.broadcasted_iota(jnp.int32, (D, D), 0)
        c = lax.broadcasted_iota(jnp.int32, (D, D), 1)
        bd = (r // KR == c // KR).astype(jnp.float32)
        kv_bd[...] *= bd
        diag_ks = (r == c).astype(jnp.float32) * ks_acc[0:1, :]
        ks_bd[...] = jax.lax.dot_general(
            diag_ks, bd, (((1,), (0,)), ((), ())),
            preferred_element_type=jnp.float32)

    @pl.when(p == 1)
    def _out():
        q = _bdot(h, wq_ref[...])
        pq = _elu1(q)
        num = _bdot(pq, kv_bd[...])
        den = _bdot(pq, ks_bd[...])
        attn_v = num / (den + 1e-6)
        x1 = x_blk + _bdot(attn_v, wo_ref[...])
        x1_ref[...] = x1
        t = _ln(x1, g2_ref[...], b2_ref[...])
        t_ref[...] = t
        logits_ref[...] = t @ wr_ref[...]


def _attention(x2d, Wq, Wk, Wv, Wo, g1, b1, g2, b2, Wr):
    full = lambda shape: pl.BlockSpec(
        shape, lambda p, bb: tuple(0 for _ in shape))
    outm = lambda p, bb: (bb * p, 0)
    return pl.pallas_call(
        _attn_kernel,
        grid=(2, NB),
        in_specs=[
            pl.BlockSpec((SB, D), lambda p, bb: (bb, 0)),
            full((D, D)), full((D, D)), full((D, D)), full((D, D)),
            full((1, D)), full((1, D)), full((1, D)), full((1, D)),
            full((D, E)),
        ],
        out_specs=[
            pl.BlockSpec((SB, D), outm),
            pl.BlockSpec((SB, D), outm),
            pl.BlockSpec((SB, E), outm),
        ],
        out_shape=[
            jax.ShapeDtypeStruct((S, D), jnp.float32),
            jax.ShapeDtypeStruct((S, D), jnp.float32),
            jax.ShapeDtypeStruct((S, E), jnp.float32),
        ],
        scratch_shapes=[
            pltpu.VMEM((D, D), jnp.float32),
            pltpu.VMEM((D, D), jnp.float32),
            pltpu.VMEM((8, D), jnp.float32),
        ],
        compiler_params=pltpu.CompilerParams(
            dimension_semantics=("arbitrary", "arbitrary")),
    )(x2d, Wq, Wk, Wv, Wo, g1.reshape(1, D), b1.reshape(1, D),
      g2.reshape(1, D), b2.reshape(1, D), Wr)


def _router_kernel(logits_ref, tv0_ref, tv1_ref, pos0_ref, pos1_ref,
                   meta_ref):
    logits = logits_ref[...]
    m = jnp.max(logits, axis=1, keepdims=True)
    ex = jnp.exp(logits - m)
    p = ex / jnp.sum(ex, axis=1, keepdims=True)
    iota = lax.broadcasted_iota(jnp.int32, (S, E), 1)
    m0 = jnp.max(p, axis=1, keepdims=True)
    i0 = jnp.min(jnp.where(p == m0, iota, E), axis=1, keepdims=True)
    p1 = jnp.where(iota == i0, -1.0, p)
    m1 = jnp.max(p1, axis=1, keepdims=True)
    i1 = jnp.min(jnp.where(p1 == m1, iota, E), axis=1, keepdims=True)
    denom = m0 + m1
    tv0_ref[...] = m0 / denom
    tv1_ref[...] = m1 / denom

    oh0 = (iota == i0).astype(jnp.float32)
    oh1 = (iota == i1).astype(jnp.float32)
    r = lax.broadcasted_iota(jnp.int32, (CSB, CSB), 0)
    c = lax.broadcasted_iota(jnp.int32, (CSB, CSB), 1)
    ltri = (r > c).astype(jnp.float32)

    carry = jnp.zeros((1, E), jnp.float32)
    ranks = []
    for oh in (oh0, oh1):
        for bb in range(S // CSB):
            blk = oh[bb * CSB:(bb + 1) * CSB]
            ex_pre = jax.lax.dot_general(
                ltri, blk, (((1,), (0,)), ((), ()))) + carry
            ranks.append(jnp.sum(ex_pre * blk, axis=1, keepdims=True))
            carry = carry + jnp.sum(blk, axis=0, keepdims=True)
    cnt = carry
    cnt_i = cnt.astype(jnp.int32)
    pcnt_i = ((cnt_i + (BLKG - 1)) // BLKG) * BLKG
    pcnt = pcnt_i.astype(jnp.float32)
    re8 = lax.broadcasted_iota(jnp.int32, (E, E), 0)
    ce8 = lax.broadcasted_iota(jnp.int32, (E, E), 1)
    ltri8 = (re8 < ce8).astype(jnp.float32)
    off = jax.lax.dot_general(pcnt, ltri8, (((1,), (0,)), ((), ())))

    rank0 = jnp.concatenate(ranks[:S // CSB], axis=0)
    rank1 = jnp.concatenate(ranks[S // CSB:], axis=0)
    offg0 = jnp.sum(oh0 * off, axis=1, keepdims=True)
    offg1 = jnp.sum(oh1 * off, axis=1, keepdims=True)
    pos0_ref[...] = (offg0 + rank0).astype(jnp.int32)
    pos1_ref[...] = (offg1 + rank1).astype(jnp.int32)

    total_i = jnp.sum(pcnt).astype(jnp.int32)
    gstart = jnp.minimum(
        lax.broadcasted_iota(jnp.int32, (8, 128), 1) * BLKG,
        total_i - BLKG)
    acc = jnp.zeros((8, 128), jnp.int32)
    for e in range(E):
        sel = (lax.broadcasted_iota(jnp.int32, (1, E), 1) == e).astype(
            jnp.float32)
        off_e = jnp.sum(off * sel).astype(jnp.int32)
        pcnt_e = jnp.sum(pcnt * sel).astype(jnp.int32)
        in_e = (gstart >= off_e) & (gstart < off_e + pcnt_e)
        acc = acc + e * in_e.astype(jnp.int32)
    nact = (jnp.sum(pcnt) / BLKG).astype(jnp.int32)
    col = lax.broadcasted_iota(jnp.int32, (8, 128), 1)
    meta_ref[...] = jnp.where(col == MAXB, nact, acc)


def _router(logits):
    return pl.pallas_call(
        _router_kernel,
        grid=(1,),
        in_specs=[
            pl.BlockSpec((S, E), lambda i: (0, 0)),
        ],
        out_specs=[
            pl.BlockSpec((S, 1), lambda i: (0, 0)),
            pl.BlockSpec((S, 1), lambda i: (0, 0)),
            pl.BlockSpec((S, 1), lambda i: (0, 0)),
            pl.BlockSpec((S, 1), lambda i: (0, 0)),
            pl.BlockSpec((8, 128), lambda i: (0, 0)),
        ],
        out_shape=[
            jax.ShapeDtypeStruct((S, 1), jnp.float32),
            jax.ShapeDtypeStruct((S, 1), jnp.float32),
            jax.ShapeDtypeStruct((S, 1), jnp.int32),
            jax.ShapeDtypeStruct((S, 1), jnp.int32),
            jax.ShapeDtypeStruct((8, 128), jnp.int32),
        ],
    )(logits)


_TOK_PER_TILE = S // 32


@functools.lru_cache(maxsize=None)
def _sc_kernels():
    mesh = plsc.VectorSubcoreMesh(core_axis_name="c", subcore_axis_name="s")

    @functools.partial(
        pl.kernel, mesh=mesh,
        out_type=jax.ShapeDtypeStruct((P, D), jnp.float32),
        scratch_types=[
            pltpu.VMEM((_TOK_PER_TILE,), jnp.int32),
            pltpu.VMEM((_TOK_PER_TILE,), jnp.int32),
            pltpu.VMEM((_TOK_PER_TILE, D), jnp.float32),
        ],
    )
    def sc_dispatch(t_hbm, pos0_hbm, pos1_hbm, out_hbm,
                    idx0_v, idx1_v, rows_v):
        wid = lax.axis_index("s") * 2 + lax.axis_index("c")
        lo = wid * _TOK_PER_TILE
        sl = pl.ds(lo, _TOK_PER_TILE)
        pltpu.sync_copy(pos0_hbm.at[sl], idx0_v)
        pltpu.sync_copy(pos1_hbm.at[sl], idx1_v)
        pltpu.sync_copy(t_hbm.at[sl], rows_v)
        pltpu.sync_copy(rows_v, out_hbm.at[idx0_v])
        pltpu.sync_copy(rows_v, out_hbm.at[idx1_v])

    @functools.partial(
        pl.kernel, mesh=mesh,
        out_type=jax.ShapeDtypeStruct((2, S, D), jnp.float32),
        scratch_types=[
            pltpu.VMEM((_TOK_PER_TILE,), jnp.int32),
            pltpu.VMEM((_TOK_PER_TILE, D), jnp.float32),
            pltpu.SemaphoreType.DMA,
        ],
    )
    def sc_gather_pairs(po_hbm, pos0_hbm, pos1_hbm, out_hbm, idx_v, rows_v,
                        sem):
        wid = lax.axis_index("s") * 2 + lax.axis_index("c")
        lo = wid * _TOK_PER_TILE
        for k, pos_hbm in ((0, pos0_hbm), (1, pos1_hbm)):
            pltpu.sync_copy(pos_hbm.at[pl.ds(lo, _TOK_PER_TILE)], idx_v)
            pltpu.async_copy(po_hbm.at[idx_v], rows_v, sem).wait()
            pltpu.sync_copy(rows_v, out_hbm.at[k].at[pl.ds(lo, _TOK_PER_TILE)])

    return sc_dispatch, sc_gather_pairs


def _sc_dispatch(t, pos0, pos1):
    return _sc_kernels()[0](t, pos0, pos1)


def _sc_gather_pairs(packed_out, pos0, pos1):
    return _sc_kernels()[1](packed_out, pos0, pos1)


def _combine_kernel(x1_ref, g0_ref, g1_ref, tv0_ref, tv1_ref, out_ref):
    out_ref[...] = (x1_ref[...] + tv0_ref[...] * g0_ref[0]
                    + tv1_ref[...] * g1_ref[0])


def _combine(x1, gathered, tv0, tv1):
    return pl.pallas_call(
        _combine_kernel,
        grid=(NQ,),
        in_specs=[
            pl.BlockSpec((TB, D), lambda q: (q, 0)),
            pl.BlockSpec((1, TB, D), lambda q: (0, q, 0)),
            pl.BlockSpec((1, TB, D), lambda q: (1, q, 0)),
            pl.BlockSpec((TB, 1), lambda q: (q, 0)),
            pl.BlockSpec((TB, 1), lambda q: (q, 0)),
        ],
        out_specs=pl.BlockSpec((TB, D), lambda q: (q, 0)),
        out_shape=jax.ShapeDtypeStruct((S, D), jnp.float32),
        compiler_params=pltpu.CompilerParams(
            dimension_semantics=("arbitrary",)),
    )(x1, gathered, gathered, tv0, tv1)


def _gmm_kernel(meta_ref, pt_ref, w1_ref, b1_ref, w2_ref, b2_ref,
                out_ref):
    g = pl.program_id(0)

    @pl.when(g < meta_ref[MAXB])
    def _compute():
        h1 = jax.nn.gelu(_bdot(pt_ref[...], w1_ref[0]) + b1_ref[0])
        out_ref[...] = _bdot(h1, w2_ref[0]) + b2_ref[0]


def _grouped_mm(meta_row, packed_t, W1, b1, W2, b2):
    grid_spec = pltpu.PrefetchScalarGridSpec(
        num_scalar_prefetch=1,
        grid=(MAXB,),
        in_specs=[
            pl.BlockSpec((BLKG, D),
                         lambda g, mr: (jnp.minimum(g, mr[MAXB] - 1), 0)),
            pl.BlockSpec((1, D, DFF), lambda g, mr: (mr[g], 0, 0)),
            pl.BlockSpec((1, 1, DFF), lambda g, mr: (mr[g], 0, 0)),
            pl.BlockSpec((1, DFF, D), lambda g, mr: (mr[g], 0, 0)),
            pl.BlockSpec((1, 1, D), lambda g, mr: (mr[g], 0, 0)),
        ],
        out_specs=pl.BlockSpec((BLKG, D), lambda g, mr: (g, 0)),
    )
    return pl.pallas_call(
        _gmm_kernel,
        grid_spec=grid_spec,
        out_shape=jax.ShapeDtypeStruct((P, D), jnp.float32),
        compiler_params=pltpu.CompilerParams(
            dimension_semantics=("arbitrary",)),
    )(meta_row, packed_t, W1,
      b1.reshape(E, 1, DFF), W2, b2.reshape(E, 1, D))


@jax.jit
def kernel(x, Wq, Wk, Wv, Wo, ln1_g, ln1_b, ln2_g, ln2_b, Wr, W1, b1, W2, b2):
    x2d = x.reshape(S, D)
    x1, t, logits = _attention(x2d, Wq, Wk, Wv, Wo, ln1_g, ln1_b,
                               ln2_g, ln2_b, Wr)
    tv0, tv1, pos0, pos1, meta = _router(logits)
    pos0 = pos0.reshape(S)
    pos1 = pos1.reshape(S)
    packed_t = _sc_dispatch(t, pos0, pos1)
    packed_out = _grouped_mm(meta[0], packed_t, W1, b1, W2, b2)
    gathered = _sc_gather_pairs(packed_out, pos0, pos1)
    out = _combine(x1, gathered, tv0, tv1)
    return out.reshape(B, S, D)


# speedup vs baseline: 1.0884x; 1.0014x over previous
import functools

import jax
import jax.numpy as jnp
from jax import lax
from jax.experimental import pallas as pl
from jax.experimental.pallas import tpu as pltpu
from jax.experimental.pallas import tpu_sc as plsc

B, S, D = 1, 2048, 768
H, KR = 12, 64
E, TOPK, DFF = 8, 2, 1536

SB = 256
TB = 512
NB = S // SB
NQ = S // TB

NPAIR = S * TOPK
BLKG = 256
MAXB = (NPAIR + E * (BLKG - 1)) // BLKG
P = 6144
CSB = 512


def _ln(h, g, b):
    mu = jnp.mean(h, axis=-1, keepdims=True)
    var = jnp.mean((h - mu) ** 2, axis=-1, keepdims=True)
    return (h - mu) * jax.lax.rsqrt(var + 1e-5) * g + b


def _elu1(x):
    return jnp.where(x > 0, x + 1.0, jnp.exp(x))


def _bdot(a, b):
    return jnp.dot(a.astype(jnp.bfloat16), b.astype(jnp.bfloat16),
                   preferred_element_type=jnp.float32)


def _bdot_t(a, b):
    return jax.lax.dot_general(
        a.astype(jnp.bfloat16), b.astype(jnp.bfloat16),
        dimension_numbers=(((0,), (0,)), ((), ())),
        preferred_element_type=jnp.float32)


def _attn_kernel(x_ref, wq_ref, wk_ref, wv_ref, wo_ref,
                 g1_ref, b1_ref, g2_ref, b2_ref, wr_ref,
                 x1_ref, t_ref, logits_ref, kv_bd, ks_bd, ks_acc):
    p = pl.program_id(0)
    bb = pl.program_id(1)

    @pl.when((p == 0) & (bb == 0))
    def _init():
        kv_bd[...] = jnp.zeros_like(kv_bd)
        ks_acc[...] = jnp.zeros_like(ks_acc)

    x_blk = x_ref[...]
    h = _ln(x_blk, g1_ref[...], b1_ref[...])

    @pl.when(p == 0)
    def _acc():
        k = h @ wk_ref[...]
        v = h @ wv_ref[...]
        pk = _elu1(k)
        kv_bd[...] += jax.lax.dot_general(
            pk, v, (((0,), (0,)), ((), ())),
            preferred_element_type=jnp.float32)
        ks_acc[0:1, :] += jnp.sum(pk, axis=0, keepdims=True)

    @pl.when((p == 0) & (bb == NB - 1))
    def _fin():
        r = lax.broadcasted_iota(jnp.int32, (D, D), 0)
        c = lax.broadcasted_iota(jnp.int32, (D, D), 1)
        bd = (r // KR == c // KR).astype(jnp.float32)
        kv_bd[...] *= bd
        diag_ks = (r == c).astype(jnp.float32) * ks_acc[0:1, :]
        ks_bd[...] = jax.lax.dot_general(
            diag_ks, bd, (((1,), (0,)), ((), ())),
            preferred_element_type=jnp.float32)

    @pl.when(p == 1)
    def _out():
        q = h @ wq_ref[...]
        pq = _elu1(q)
        num = pq @ kv_bd[...]
        den = pq @ ks_bd[...]
        attn_v = num / (den + 1e-6)
        x1 = x_blk + attn_v @ wo_ref[...]
        x1_ref[...] = x1
        t = _ln(x1, g2_ref[...], b2_ref[...])
        t_ref[...] = t
        logits_ref[...] = t @ wr_ref[...]


def _attention(x2d, Wq, Wk, Wv, Wo, g1, b1, g2, b2, Wr):
    full = lambda shape: pl.BlockSpec(
        shape, lambda p, bb: tuple(0 for _ in shape))
    outm = lambda p, bb: (bb * p, 0)
    return pl.pallas_call(
        _attn_kernel,
        grid=(2, NB),
        in_specs=[
            pl.BlockSpec((SB, D), lambda p, bb: (bb, 0)),
            full((D, D)), full((D, D)), full((D, D)), full((D, D)),
            full((1, D)), full((1, D)), full((1, D)), full((1, D)),
            full((D, E)),
        ],
        out_specs=[
            pl.BlockSpec((SB, D), outm),
            pl.BlockSpec((SB, D), outm),
            pl.BlockSpec((SB, E), outm),
        ],
        out_shape=[
            jax.ShapeDtypeStruct((S, D), jnp.float32),
            jax.ShapeDtypeStruct((S, D), jnp.float32),
            jax.ShapeDtypeStruct((S, E), jnp.float32),
        ],
        scratch_shapes=[
            pltpu.VMEM((D, D), jnp.float32),
            pltpu.VMEM((D, D), jnp.float32),
            pltpu.VMEM((8, D), jnp.float32),
        ],
        compiler_params=pltpu.CompilerParams(
            dimension_semantics=("arbitrary", "arbitrary")),
    )(x2d, Wq, Wk, Wv, Wo, g1.reshape(1, D), b1.reshape(1, D),
      g2.reshape(1, D), b2.reshape(1, D), Wr)


def _router_kernel(logits_ref, tv0_ref, tv1_ref, pos0_ref, pos1_ref,
                   meta_ref):
    logits = logits_ref[...]
    m = jnp.max(logits, axis=1, keepdims=True)
    ex = jnp.exp(logits - m)
    p = ex / jnp.sum(ex, axis=1, keepdims=True)
    iota = lax.broadcasted_iota(jnp.int32, (S, E), 1)
    m0 = jnp.max(p, axis=1, keepdims=True)
    i0 = jnp.min(jnp.where(p == m0, iota, E), axis=1, keepdims=True)
    p1 = jnp.where(iota == i0, -1.0, p)
    m1 = jnp.max(p1, axis=1, keepdims=True)
    i1 = jnp.min(jnp.where(p1 == m1, iota, E), axis=1, keepdims=True)
    denom = m0 + m1
    tv0_ref[...] = m0 / denom
    tv1_ref[...] = m1 / denom

    oh0 = (iota == i0).astype(jnp.float32)
    oh1 = (iota == i1).astype(jnp.float32)
    r = lax.broadcasted_iota(jnp.int32, (CSB, CSB), 0)
    c = lax.broadcasted_iota(jnp.int32, (CSB, CSB), 1)
    ltri = (r > c).astype(jnp.float32)

    carry = jnp.zeros((1, E), jnp.float32)
    ranks = []
    for oh in (oh0, oh1):
        for bb in range(S // CSB):
            blk = oh[bb * CSB:(bb + 1) * CSB]
            ex_pre = jax.lax.dot_general(
                ltri, blk, (((1,), (0,)), ((), ()))) + carry
            ranks.append(jnp.sum(ex_pre * blk, axis=1, keepdims=True))
            carry = carry + jnp.sum(blk, axis=0, keepdims=True)
    cnt = carry
    cnt_i = cnt.astype(jnp.int32)
    pcnt_i = ((cnt_i + (BLKG - 1)) // BLKG) * BLKG
    pcnt = pcnt_i.astype(jnp.float32)
    re8 = lax.broadcasted_iota(jnp.int32, (E, E), 0)
    ce8 = lax.broadcasted_iota(jnp.int32, (E, E), 1)
    ltri8 = (re8 < ce8).astype(jnp.float32)
    off = jax.lax.dot_general(pcnt, ltri8, (((1,), (0,)), ((), ())))

    rank0 = jnp.concatenate(ranks[:S // CSB], axis=0)
    rank1 = jnp.concatenate(ranks[S // CSB:], axis=0)
    offg0 = jnp.sum(oh0 * off, axis=1, keepdims=True)
    offg1 = jnp.sum(oh1 * off, axis=1, keepdims=True)
    pos0_ref[...] = (offg0 + rank0).astype(jnp.int32)
    pos1_ref[...] = (offg1 + rank1).astype(jnp.int32)

    total_i = jnp.sum(pcnt).astype(jnp.int32)
    gstart = jnp.minimum(
        lax.broadcasted_iota(jnp.int32, (8, 128), 1) * BLKG,
        total_i - BLKG)
    acc = jnp.zeros((8, 128), jnp.int32)
    for e in range(E):
        sel = (lax.broadcasted_iota(jnp.int32, (1, E), 1) == e).astype(
            jnp.float32)
        off_e = jnp.sum(off * sel).astype(jnp.int32)
        pcnt_e = jnp.sum(pcnt * sel).astype(jnp.int32)
        in_e = (gstart >= off_e) & (gstart < off_e + pcnt_e)
        acc = acc + e * in_e.astype(jnp.int32)
    nact = (jnp.sum(pcnt) / BLKG).astype(jnp.int32)
    col = lax.broadcasted_iota(jnp.int32, (8, 128), 1)
    meta_ref[...] = jnp.where(col == MAXB, nact, acc)


def _router(logits):
    return pl.pallas_call(
        _router_kernel,
        grid=(1,),
        in_specs=[
            pl.BlockSpec((S, E), lambda i: (0, 0)),
        ],
        out_specs=[
            pl.BlockSpec((S, 1), lambda i: (0, 0)),
            pl.BlockSpec((S, 1), lambda i: (0, 0)),
            pl.BlockSpec((S, 1), lambda i: (0, 0)),
            pl.BlockSpec((S, 1), lambda i: (0, 0)),
            pl.BlockSpec((8, 128), lambda i: (0, 0)),
        ],
        out_shape=[
            jax.ShapeDtypeStruct((S, 1), jnp.float32),
            jax.ShapeDtypeStruct((S, 1), jnp.float32),
            jax.ShapeDtypeStruct((S, 1), jnp.int32),
            jax.ShapeDtypeStruct((S, 1), jnp.int32),
            jax.ShapeDtypeStruct((8, 128), jnp.int32),
        ],
    )(logits)


_TOK_PER_TILE = S // 32


@functools.lru_cache(maxsize=None)
def _sc_kernels():
    mesh = plsc.VectorSubcoreMesh(core_axis_name="c", subcore_axis_name="s")

    @functools.partial(
        pl.kernel, mesh=mesh,
        out_type=jax.ShapeDtypeStruct((P, D), jnp.float32),
        scratch_types=[
            pltpu.VMEM((_TOK_PER_TILE,), jnp.int32),
            pltpu.VMEM((_TOK_PER_TILE,), jnp.int32),
            pltpu.VMEM((_TOK_PER_TILE, D), jnp.float32),
        ],
    )
    def sc_dispatch(t_hbm, pos0_hbm, pos1_hbm, out_hbm,
                    idx0_v, idx1_v, rows_v):
        wid = lax.axis_index("s") * 2 + lax.axis_index("c")
        lo = wid * _TOK_PER_TILE
        sl = pl.ds(lo, _TOK_PER_TILE)
        pltpu.sync_copy(pos0_hbm.at[sl], idx0_v)
        pltpu.sync_copy(pos1_hbm.at[sl], idx1_v)
        pltpu.sync_copy(t_hbm.at[sl], rows_v)
        pltpu.sync_copy(rows_v, out_hbm.at[idx0_v])
        pltpu.sync_copy(rows_v, out_hbm.at[idx1_v])

    @functools.partial(
        pl.kernel, mesh=mesh,
        out_type=jax.ShapeDtypeStruct((2, S, D), jnp.float32),
        scratch_types=[
            pltpu.VMEM((_TOK_PER_TILE,), jnp.int32),
            pltpu.VMEM((_TOK_PER_TILE, D), jnp.float32),
            pltpu.SemaphoreType.DMA,
        ],
    )
    def sc_gather_pairs(po_hbm, pos0_hbm, pos1_hbm, out_hbm, idx_v, rows_v,
                        sem):
        wid = lax.axis_index("s") * 2 + lax.axis_index("c")
        lo = wid * _TOK_PER_TILE
        for k, pos_hbm in ((0, pos0_hbm), (1, pos1_hbm)):
            pltpu.sync_copy(pos_hbm.at[pl.ds(lo, _TOK_PER_TILE)], idx_v)
            pltpu.async_copy(po_hbm.at[idx_v], rows_v, sem).wait()
            pltpu.sync_copy(rows_v, out_hbm.at[k].at[pl.ds(lo, _TOK_PER_TILE)])

    return sc_dispatch, sc_gather_pairs


def _sc_dispatch(t, pos0, pos1):
    return _sc_kernels()[0](t, pos0, pos1)


def _sc_gather_pairs(packed_out, pos0, pos1):
    return _sc_kernels()[1](packed_out, pos0, pos1)


def _combine_kernel(x1_ref, g0_ref, g1_ref, tv0_ref, tv1_ref, out_ref):
    out_ref[...] = (x1_ref[...] + tv0_ref[...] * g0_ref[0]
                    + tv1_ref[...] * g1_ref[0])


def _combine(x1, gathered, tv0, tv1):
    return pl.pallas_call(
        _combine_kernel,
        grid=(NQ,),
        in_specs=[
            pl.BlockSpec((TB, D), lambda q: (q, 0)),
            pl.BlockSpec((1, TB, D), lambda q: (0, q, 0)),
            pl.BlockSpec((1, TB, D), lambda q: (1, q, 0)),
            pl.BlockSpec((TB, 1), lambda q: (q, 0)),
            pl.BlockSpec((TB, 1), lambda q: (q, 0)),
        ],
        out_specs=pl.BlockSpec((TB, D), lambda q: (q, 0)),
        out_shape=jax.ShapeDtypeStruct((S, D), jnp.float32),
        compiler_params=pltpu.CompilerParams(
            dimension_semantics=("arbitrary",)),
    )(x1, gathered, gathered, tv0, tv1)


def _gmm_kernel(meta_ref, pt_ref, w1_ref, b1_ref, w2_ref, b2_ref,
                out_ref):
    g = pl.program_id(0)

    @pl.when(g < meta_ref[MAXB])
    def _compute():
        h1 = jax.nn.gelu(_bdot(pt_ref[...], w1_ref[0]) + b1_ref[0])
        out_ref[...] = _bdot(h1, w2_ref[0]) + b2_ref[0]


def _grouped_mm(meta_row, packed_t, W1, b1, W2, b2):
    grid_spec = pltpu.PrefetchScalarGridSpec(
        num_scalar_prefetch=1,
        grid=(MAXB,),
        in_specs=[
            pl.BlockSpec((BLKG, D),
                         lambda g, mr: (jnp.minimum(g, mr[MAXB] - 1), 0)),
            pl.BlockSpec((1, D, DFF), lambda g, mr: (mr[g], 0, 0)),
            pl.BlockSpec((1, 1, DFF), lambda g, mr: (mr[g], 0, 0)),
            pl.BlockSpec((1, DFF, D), lambda g, mr: (mr[g], 0, 0)),
            pl.BlockSpec((1, 1, D), lambda g, mr: (mr[g], 0, 0)),
        ],
        out_specs=pl.BlockSpec((BLKG, D), lambda g, mr: (g, 0)),
    )
    return pl.pallas_call(
        _gmm_kernel,
        grid_spec=grid_spec,
        out_shape=jax.ShapeDtypeStruct((P, D), jnp.float32),
        compiler_params=pltpu.CompilerParams(
            dimension_semantics=("arbitrary",)),
    )(meta_row, packed_t, W1,
      b1.reshape(E, 1, DFF), W2, b2.reshape(E, 1, D))


@jax.jit
def kernel(x, Wq, Wk, Wv, Wo, ln1_g, ln1_b, ln2_g, ln2_b, Wr, W1, b1, W2, b2):
    x2d = x.reshape(S, D)
    x1, t, logits = _attention(x2d, Wq, Wk, Wv, Wo, ln1_g, ln1_b,
                               ln2_g, ln2_b, Wr)
    tv0, tv1, pos0, pos1, meta = _router(logits)
    pos0 = pos0.reshape(S)
    pos1 = pos1.reshape(S)
    packed_t = _sc_dispatch(t, pos0, pos1)
    packed_out = _grouped_mm(meta[0], packed_t, W1, b1, W2, b2)
    gathered = _sc_gather_pairs(packed_out, pos0, pos1)
    out = _combine(x1, gathered, tv0, tv1)
    return out.reshape(B, S, D)
